# baseline (speedup 1.0000x reference)
import functools

import jax
import jax.numpy as jnp
from jax.experimental import pallas as pl

N = 4096
D = 256
TILE = 512


def _dot(a, b):
    return jax.lax.dot_general(a.astype(jnp.bfloat16), b.astype(jnp.bfloat16),
                               (((1,), (0,)), ((), ())),
                               preferred_element_type=jnp.float32)


def _phi_kernel(x_ref, w0_ref, b0_ref, w1_ref, b1_ref, h_ref):
    t = jnp.maximum(_dot(x_ref[...], w0_ref[...]) + b0_ref[...], 0.0)
    h_ref[...] = _dot(t, w1_ref[...]) + b1_ref[...]


def _agg_kernel(adj_ref, h_ref, out_ref):
    out_ref[...] = _dot(adj_ref[...], h_ref[...])


def _tail_kernel(adj_d_ref, adj_s_ref, na_ref, x_ref,
                 p1w0_ref, p1b0_ref, p1w1_ref, p1b1_ref,
                 p2w0_ref, p2b0_ref, p2w1_ref, p2b1_ref,
                 mw0x_ref, mw0d_ref, mw0s_ref, mb0_ref, mw1_ref, mb1_ref,
                 out_ref):
    na = na_ref[...]
    t1 = _dot(adj_d_ref[...], na)
    t2 = _dot(adj_s_ref[...], na)
    hd = jnp.maximum(_dot(t1, p1w0_ref[...]) + p1b0_ref[...], 0.0)
    hd = _dot(hd, p1w1_ref[...]) + p1b1_ref[...]
    hs = jnp.maximum(_dot(t2, p2w0_ref[...]) + p2b0_ref[...], 0.0)
    hs = _dot(hs, p2w1_ref[...]) + p2b1_ref[...]
    u = (_dot(x_ref[...], mw0x_ref[...]) + _dot(hd, mw0d_ref[...])
         + _dot(hs, mw0s_ref[...]) + mb0_ref[...])
    u = jnp.maximum(u, 0.0)
    out_ref[...] = _dot(u, mw1_ref[...]) + mb1_ref[...]


def _row_spec(width):
    return pl.BlockSpec((TILE, width), lambda i: (i, 0))


def _full_spec(shape):
    return pl.BlockSpec(shape, lambda i: (0,) * len(shape))


@jax.jit
def kernel(net_inst_adj, inst_net_adj_v_drive, inst_net_adj_v_sink, x,
           phi_w0, phi_b0, phi_w1, phi_b1,
           psi1_w0, psi1_b0, psi1_w1, psi1_b1,
           psi2_w0, psi2_b0, psi2_w1, psi2_b1,
           mlp_w0, mlp_b0, mlp_w1, mlp_b1):
    grid = (N // TILE,)
    fo32 = jax.ShapeDtypeStruct((N, D), jnp.float32)

    phi_b0_2d = phi_b0.reshape(1, D)
    phi_b1_2d = phi_b1.reshape(1, D)

    h = pl.pallas_call(
        _phi_kernel,
        grid=grid,
        in_specs=[_row_spec(D), _full_spec((D, D)), _full_spec((1, D)),
                  _full_spec((D, D)), _full_spec((1, D))],
        out_specs=_row_spec(D),
        out_shape=fo32,
    )(x, phi_w0, phi_b0_2d, phi_w1, phi_b1_2d)

    net_agg = pl.pallas_call(
        _agg_kernel,
        grid=grid,
        in_specs=[_row_spec(N), _full_spec((N, D))],
        out_specs=_row_spec(D),
        out_shape=fo32,
    )(net_inst_adj, h)

    mw0x = mlp_w0[0:D]
    mw0d = mlp_w0[D:2 * D]
    mw0s = mlp_w0[2 * D:3 * D]

    out = pl.pallas_call(
        _tail_kernel,
        grid=grid,
        in_specs=[_row_spec(N), _row_spec(N), _full_spec((N, D)),
                  _row_spec(D),
                  _full_spec((D, D)), _full_spec((1, D)),
                  _full_spec((D, D)), _full_spec((1, D)),
                  _full_spec((D, D)), _full_spec((1, D)),
                  _full_spec((D, D)), _full_spec((1, D)),
                  _full_spec((D, 3 * D)), _full_spec((D, 3 * D)),
                  _full_spec((D, 3 * D)), _full_spec((1, 3 * D)),
                  _full_spec((3 * D, D)), _full_spec((1, D))],
        out_specs=_row_spec(D),
        out_shape=fo32,
    )(inst_net_adj_v_drive, inst_net_adj_v_sink, net_agg, x,
      psi1_w0, psi1_b0.reshape(1, D), psi1_w1, psi1_b1.reshape(1, D),
      psi2_w0, psi2_b0.reshape(1, D), psi2_w1, psi2_b1.reshape(1, D),
      mw0x, mw0d, mw0s, mlp_b0.reshape(1, 3 * D),
      mlp_w1, mlp_b1.reshape(1, D))
    return out


# speedup vs baseline: 1.0212x; 1.0212x over previous
import jax
import jax.numpy as jnp
from jax.experimental import pallas as pl

N = 4096
D = 256
TILE_AGG = 1024
TILE = 512


def _dot(a, b):
    return jax.lax.dot_general(a.astype(jnp.bfloat16), b.astype(jnp.bfloat16),
                               (((1,), (0,)), ((), ())),
                               preferred_element_type=jnp.float32)


def _phi_kernel(x_ref, w0_ref, b0_ref, w1_ref, b1_ref, h_ref):
    t = jnp.maximum(_dot(x_ref[...], w0_ref[...]) + b0_ref[...], 0.0)
    h_ref[...] = (_dot(t, w1_ref[...]) + b1_ref[...]).astype(jnp.bfloat16)


def _agg_kernel(adj_ref, h_ref, out_ref):
    out_ref[...] = _dot(adj_ref[...], h_ref[...]).astype(jnp.bfloat16)


def _tail_kernel(adj_d_ref, adj_s_ref, na_ref, x_ref,
                 p1w0_ref, p1b0_ref, p1w1_ref, p1b1_ref,
                 p2w0_ref, p2b0_ref, p2w1_ref, p2b1_ref,
                 mw0x_ref, mw0d_ref, mw0s_ref, mb0_ref, mw1_ref, mb1_ref,
                 out_ref):
    na = na_ref[...]
    t1 = _dot(adj_d_ref[...], na)
    t2 = _dot(adj_s_ref[...], na)
    hd = jnp.maximum(_dot(t1, p1w0_ref[...]) + p1b0_ref[...], 0.0)
    hd = _dot(hd, p1w1_ref[...]) + p1b1_ref[...]
    hs = jnp.maximum(_dot(t2, p2w0_ref[...]) + p2b0_ref[...], 0.0)
    hs = _dot(hs, p2w1_ref[...]) + p2b1_ref[...]
    u = (_dot(x_ref[...], mw0x_ref[...]) + _dot(hd, mw0d_ref[...])
         + _dot(hs, mw0s_ref[...]) + mb0_ref[...])
    u = jnp.maximum(u, 0.0)
    out_ref[...] = _dot(u, mw1_ref[...]) + mb1_ref[...]


def _row_spec(tile, width):
    return pl.BlockSpec((tile, width), lambda i: (i, 0))


def _full_spec(shape):
    return pl.BlockSpec(shape, lambda i: (0,) * len(shape))


@jax.jit
def kernel(net_inst_adj, inst_net_adj_v_drive, inst_net_adj_v_sink, x,
           phi_w0, phi_b0, phi_w1, phi_b1,
           psi1_w0, psi1_b0, psi1_w1, psi1_b1,
           psi2_w0, psi2_b0, psi2_w1, psi2_b1,
           mlp_w0, mlp_b0, mlp_w1, mlp_b1):
    bf16 = jax.ShapeDtypeStruct((N, D), jnp.bfloat16)

    h = pl.pallas_call(
        _phi_kernel,
        grid=(N // TILE,),
        in_specs=[_row_spec(TILE, D), _full_spec((D, D)), _full_spec((1, D)),
                  _full_spec((D, D)), _full_spec((1, D))],
        out_specs=_row_spec(TILE, D),
        out_shape=bf16,
    )(x, phi_w0, phi_b0.reshape(1, D), phi_w1, phi_b1.reshape(1, D))

    net_agg = pl.pallas_call(
        _agg_kernel,
        grid=(N // TILE_AGG,),
        in_specs=[_row_spec(TILE_AGG, N), _full_spec((N, D))],
        out_specs=_row_spec(TILE_AGG, D),
        out_shape=bf16,
    )(net_inst_adj, h)

    mw0x = mlp_w0[0:D]
    mw0d = mlp_w0[D:2 * D]
    mw0s = mlp_w0[2 * D:3 * D]

    out = pl.pallas_call(
        _tail_kernel,
        grid=(N // TILE,),
        in_specs=[_row_spec(TILE, N), _row_spec(TILE, N), _full_spec((N, D)),
                  _row_spec(TILE, D),
                  _full_spec((D, D)), _full_spec((1, D)),
                  _full_spec((D, D)), _full_spec((1, D)),
                  _full_spec((D, D)), _full_spec((1, D)),
                  _full_spec((D, D)), _full_spec((1, D)),
                  _full_spec((D, 3 * D)), _full_spec((D, 3 * D)),
                  _full_spec((D, 3 * D)), _full_spec((1, 3 * D)),
                  _full_spec((3 * D, D)), _full_spec((1, D))],
        out_specs=_row_spec(TILE, D),
        out_shape=jax.ShapeDtypeStruct((N, D), jnp.float32),
    )(inst_net_adj_v_drive, inst_net_adj_v_sink, net_agg, x,
      psi1_w0, psi1_b0.reshape(1, D), psi1_w1, psi1_b1.reshape(1, D),
      psi2_w0, psi2_b0.reshape(1, D), psi2_w1, psi2_b1.reshape(1, D),
      mw0x, mw0d, mw0s, mlp_b0.reshape(1, 3 * D),
      mlp_w1, mlp_b1.reshape(1, D))
    return out
